# baseline (speedup 1.0000x reference)
import functools

import jax
import jax.numpy as jnp
from jax import lax
from jax.experimental import pallas as pl
from jax.experimental.pallas import tpu as pltpu
from jax.experimental.pallas import tpu_sc as plsc

N = 10000
E = 320000
NPAD = 10240
SENT = 10000
NG = 64
H = 128
NC = 2
NS = 16
NW = NC * NS
ECT = 10240
NCH_OUT = ECT // 16
NROW = NPAD // 128


BN = 1024


def _mm_body(g_ref, w_ref, deg_ref, hs_ref, dinv_ref):
    degv = deg_ref[...]
    deg = degv[0] + degv[1] + 1.0
    dinv = lax.rsqrt(deg)
    gs = g_ref[...] * dinv
    hs_ref[...] = jnp.dot(gs, w_ref[...], preferred_element_type=jnp.float32)
    dinv_ref[...] = dinv


def _mm(g, w, deg2):
    return pl.pallas_call(
        _mm_body,
        grid=(NPAD // BN,),
        in_specs=[
            pl.BlockSpec((BN, H), lambda i: (i, 0)),
            pl.BlockSpec((H, H), lambda i: (0, 0)),
            pl.BlockSpec((2, BN, 1), lambda i: (0, i, 0)),
        ],
        out_specs=[
            pl.BlockSpec((BN, H), lambda i: (i, 0)),
            pl.BlockSpec((BN, 1), lambda i: (i, 0)),
        ],
        out_shape=[
            jax.ShapeDtypeStruct((NPAD, H), jnp.float32),
            jax.ShapeDtypeStruct((NPAD, 1), jnp.float32),
        ],
    )(g, w, deg2)


def _epi_body(agg_ref, hs_ref, dinv_ref, b_ref, p_ref, sel_ref, gn_ref, key_ref):
    aggv = agg_ref[...]
    hs = hs_ref[...]
    dinv = dinv_ref[...]
    h = jax.nn.relu(dinv * (aggv[0] + aggv[1] + hs) + b_ref[...])
    pc = p_ref[...]
    norm = jnp.sqrt(jnp.sum(pc * pc))
    sc = jnp.tanh(jnp.dot(h, pc, preferred_element_type=jnp.float32) / (norm + 1e-12))
    gn_ref[...] = h * sc
    key_ref[...] = jnp.where(sel_ref[...] > 0, sc, -2.0)


def _epi(agg2, hs, dinv, b2d, pcol, selrow):
    return pl.pallas_call(
        _epi_body,
        grid=(NPAD // BN,),
        in_specs=[
            pl.BlockSpec((2, BN, H), lambda i: (0, i, 0)),
            pl.BlockSpec((BN, H), lambda i: (i, 0)),
            pl.BlockSpec((BN, 1), lambda i: (i, 0)),
            pl.BlockSpec((1, H), lambda i: (0, 0)),
            pl.BlockSpec((H, 1), lambda i: (0, 0)),
            pl.BlockSpec((BN, 1), lambda i: (i, 0)),
        ],
        out_specs=[
            pl.BlockSpec((BN, H), lambda i: (i, 0)),
            pl.BlockSpec((BN, 1), lambda i: (i, 0)),
        ],
        out_shape=[
            jax.ShapeDtypeStruct((NPAD, H), jnp.float32),
            jax.ShapeDtypeStruct((NPAD, 1), jnp.float32),
        ],
    )(agg2, hs, dinv, b2d, pcol, selrow)


def _make_select(k):
    def body(key_ref, sel_ref):
        keyv = key_ref[...]
        bu = lax.bitcast_convert_type(keyv, jnp.uint32)
        top = jnp.uint32(0x80000000)
        mu = jnp.where(bu >= top, jnp.bitwise_not(bu), bu | top)

        def sb(i, t):
            sh = (31 - i).astype(jnp.uint32)
            cand = t | lax.shift_left(jnp.uint32(1), sh)
            cnt = jnp.sum((mu >= cand).astype(jnp.int32))
            return jnp.where(cnt >= k, cand, t)

        t = lax.fori_loop(0, 32, sb, jnp.uint32(0))
        gt = mu > t
        tie = (mu == t).astype(jnp.float32)
        need = (k - jnp.sum(gt.astype(jnp.int32))).astype(jnp.float32)
        m128 = (lax.broadcasted_iota(jnp.int32, (128, 128), 0)
                <= lax.broadcasted_iota(jnp.int32, (128, 128), 1)).astype(jnp.float32)
        rank_in_row = jnp.dot(tie, m128, preferred_element_type=jnp.float32)
        rowsum = jnp.sum(tie, axis=1, keepdims=True)
        s80 = (lax.broadcasted_iota(jnp.int32, (NROW, NROW), 1)
               < lax.broadcasted_iota(jnp.int32, (NROW, NROW), 0)).astype(jnp.float32)
        rowoff = jnp.dot(s80, rowsum, preferred_element_type=jnp.float32)
        rank = rank_in_row + rowoff
        selv = gt | ((tie > 0.0) & (rank <= need))
        sel_ref[...] = selv.astype(jnp.int32)

    return pl.pallas_call(
        body,
        out_shape=jax.ShapeDtypeStruct((NROW, 128), jnp.int32),
    )


_select_1 = _make_select(5000)
_select_2 = _make_select(2500)
_select_3 = _make_select(1250)

NBLK = NPAD // 8


def _readout_body(g_ref, sel_ref, btt_ref, st_ref, en_ref, xl_ref, gm_s):
    neg = jnp.float32(-jnp.inf)
    sel = sel_ref[...] > 0
    g = g_ref[...]
    gm = jnp.where(sel, g, neg)
    gm_s[...] = gm
    bm = jnp.max(gm.reshape(NBLK, 8, 128), axis=1)
    stt0 = (btt_ref[...] == lax.broadcasted_iota(jnp.int32, (NG, 1), 0)).astype(jnp.float32)
    gs = jnp.where(sel, g, 0.0)
    sums = jnp.dot(stt0, gs, preferred_element_type=jnp.float32)
    counts = jnp.dot(stt0, sel.astype(jnp.float32), preferred_element_type=jnp.float32)
    ga = sums / jnp.maximum(counts, 1.0)
    xl_ref[pl.ds(0, NG), pl.ds(128, 128)] = ga
    biota = lax.broadcasted_iota(jnp.int32, (NBLK, 1), 0)
    riota8 = lax.broadcasted_iota(jnp.int32, (8, 1), 0)
    giota = lax.broadcasted_iota(jnp.int32, (NG, 1), 0)

    def gloop(gi, acc):
        s_g = st_ref[gi]
        e_g = en_ref[gi]
        bs = (s_g + 7) // 8
        be = e_g // 8
        m = (biota >= bs) & (biota < be)
        mfull = jnp.max(jnp.where(m, bm, neg), axis=0, keepdims=True)
        hb = s_g // 8
        tb = jnp.maximum(e_g - 1, 0) // 8
        hrows = gm_s[pl.ds(hb * 8, 8), :]
        hm = (riota8 + hb * 8 >= s_g) & (riota8 + hb * 8 < e_g)
        hmax = jnp.max(jnp.where(hm, hrows, neg), axis=0, keepdims=True)
        trows = gm_s[pl.ds(tb * 8, 8), :]
        tm = (riota8 + tb * 8 >= s_g) & (riota8 + tb * 8 < e_g)
        tmax = jnp.max(jnp.where(tm, trows, neg), axis=0, keepdims=True)
        mg = jnp.maximum(jnp.maximum(mfull, hmax), tmax)
        mg = jnp.where(jnp.isfinite(mg), mg, 0.0)
        return jnp.where(giota == gi, mg, acc)

    gmax = lax.fori_loop(0, NG, gloop, jnp.zeros((NG, 128), jnp.float32))
    xl_ref[pl.ds(0, NG), pl.ds(0, 128)] = gmax


def _readout(gnext, selrow, batcht, starts, ends):
    return pl.pallas_call(
        _readout_body,
        in_specs=[
            pl.BlockSpec((NPAD, H), lambda: (0, 0)),
            pl.BlockSpec((NPAD, 1), lambda: (0, 0)),
            pl.BlockSpec((1, NPAD), lambda: (0, 0)),
            pl.BlockSpec(memory_space=pltpu.SMEM),
            pl.BlockSpec(memory_space=pltpu.SMEM),
        ],
        out_shape=jax.ShapeDtypeStruct((NG, 256), jnp.float32),
        scratch_shapes=[
            pltpu.VMEM((NPAD, 128), jnp.float32),
        ],
    )(gnext, selrow, batcht, starts, ends)


def _head_body(x1_ref, x2_ref, x3_ref, ic_ref, we_ref, l1_ref, bl1_ref,
               l2_ref, bl2_ref, l3_ref, o_ref):
    hg = (jax.nn.relu(x1_ref[...]) + jax.nn.relu(x2_ref[...])
          + jax.nn.relu(x3_ref[...]))
    emb = jax.nn.relu(jnp.dot(ic_ref[...], we_ref[...], preferred_element_type=jnp.float32))
    fus = jnp.concatenate([emb, hg], axis=1)
    o = jax.nn.relu(jnp.dot(fus, l1_ref[...], preferred_element_type=jnp.float32) + bl1_ref[...])
    o = jax.nn.relu(jnp.dot(o, l2_ref[...], preferred_element_type=jnp.float32) + bl2_ref[...])
    o_ref[...] = jnp.dot(o, l3_ref[...], preferred_element_type=jnp.float32)


def _head(x1, x2, x3, inp_c, We, L1, bL1, L2, bL2, L3):
    return pl.pallas_call(
        _head_body,
        out_shape=jax.ShapeDtypeStruct((NG, 1), jnp.float32),
    )(x1, x2, x3, inp_c, We, L1, bL1.reshape(1, -1), L2, bL2.reshape(1, -1), L3)


def _make_compact_deg(ein_t):
    nch_in = ein_t // 16
    mesh = plsc.VectorSubcoreMesh(core_axis_name="c", subcore_axis_name="s")

    def body(src_hbm, dst_hbm, sel_hbm, srco_hbm, dsto_hbm, deg_hbm, cnt_hbm,
             sel_v, src_v, dst_v, srco_v, dsto_v, deg_v, idx80_v, cnt_v, deg_sh):
        c = lax.axis_index("c")
        s = lax.axis_index("s")
        wid = c * NS + s
        pltpu.sync_copy(sel_hbm, sel_v)
        pltpu.sync_copy(src_hbm.at[wid], src_v)
        pltpu.sync_copy(dst_hbm.at[wid], dst_v)
        zeros16f = jnp.zeros((16,), jnp.float32)
        sent16 = jnp.full((16,), SENT, jnp.int32)
        ones16f = jnp.ones((16,), jnp.float32)
        c127 = jnp.full((16,), 127, jnp.int32)

        def z_body(i, carry):
            deg_v[i // 8, pl.ds((i % 8) * 16, 16)] = zeros16f
            return carry
        lax.fori_loop(0, NROW * 8, z_body, 0)

        def f_body(i, carry):
            srco_v[pl.ds(i * 16, 16)] = sent16
            dsto_v[pl.ds(i * 16, 16)] = sent16
            return carry
        lax.fori_loop(0, NCH_OUT, f_body, 0)

        def i_body(j, carry):
            idx80_v[pl.ds(j * 16, 16)] = lax.iota(jnp.int32, 16) + j * 16
            return carry
        lax.fori_loop(0, NROW // 16, i_body, 0)

        def e_body(i, off):
            s16 = src_v[pl.ds(i * 16, 16)]
            d16 = dst_v[pl.ds(i * 16, 16)]
            ss = plsc.load_gather(sel_v, [lax.shift_right_logical(s16, 7), lax.bitwise_and(s16, c127)])
            dd = plsc.load_gather(sel_v, [lax.shift_right_logical(d16, 7), lax.bitwise_and(d16, c127)])
            m = (ss > 0) & (dd > 0)
            plsc.store_compressed(srco_v.at[pl.ds(off, 16)], s16, mask=m)
            plsc.store_compressed(dsto_v.at[pl.ds(off, 16)], d16, mask=m)
            r16 = lax.shift_right_logical(d16, 7)
            c16 = lax.bitwise_and(d16, c127)
            plsc.addupdate_scatter(deg_v, [r16, c16], ones16f, mask=m)
            return off + jnp.sum(m.astype(jnp.int32))
        off = lax.fori_loop(0, nch_in, e_body, jnp.int32(0))

        cnt_v[...] = jnp.full((16,), 1, jnp.int32) * off
        pltpu.sync_copy(cnt_v, cnt_hbm.at[wid])
        pltpu.sync_copy(srco_v, srco_hbm.at[wid])
        pltpu.sync_copy(dsto_v, dsto_hbm.at[wid])

        @pl.when(s == 0)
        def _():
            pltpu.sync_copy(deg_v, deg_sh)
        plsc.subcore_barrier()

        @pl.when(s != 0)
        def _():
            pltpu.sync_copy(deg_v, deg_sh.at[idx80_v], add=True)
        plsc.subcore_barrier()

        @pl.when(s == 0)
        def _():
            pltpu.sync_copy(deg_sh, deg_hbm.at[c])

    return pl.kernel(
        body,
        out_type=[
            jax.ShapeDtypeStruct((NW, ECT), jnp.int32),
            jax.ShapeDtypeStruct((NW, ECT), jnp.int32),
            jax.ShapeDtypeStruct((NC, NROW, 128), jnp.float32),
            jax.ShapeDtypeStruct((NW, 16), jnp.int32),
        ],
        mesh=mesh,
        scratch_types=[
            pltpu.VMEM((NROW, 128), jnp.int32),
            pltpu.VMEM((ein_t,), jnp.int32),
            pltpu.VMEM((ein_t,), jnp.int32),
            pltpu.VMEM((ECT,), jnp.int32),
            pltpu.VMEM((ECT,), jnp.int32),
            pltpu.VMEM((NROW, 128), jnp.float32),
            pltpu.VMEM((NROW,), jnp.int32),
            pltpu.VMEM((16,), jnp.int32),
            pltpu.VMEM_SHARED((NROW, 128), jnp.float32),
        ],
        compiler_params=pltpu.CompilerParams(needs_layout_passes=False),
    )


_compact_deg_l1 = _make_compact_deg(E // NW)
_compact_deg_l2 = _make_compact_deg(ECT)


NBUF = 2
SG = 4
NSG = ECT // (SG * 128)


def _agg_body(src_hbm, dst_hbm, cnt_hbm, hs_hbm, agg_hbm,
              srcr_v, dstr_v, cnt_v, rows0, rows1, agg_sh,
              sem_s0, sem_s1, sem0, sem1, sem_z):
    c = lax.axis_index("c")
    s = lax.axis_index("s")
    wid = c * NS + s
    rows = (rows0, rows1)
    sems = (sem0, sem1)
    ssems = (sem_s0, sem_s1)
    SGW = SG * 128
    d_s0 = pltpu.async_copy(src_hbm.at[wid, pl.ds(0, SGW)],
                            srcr_v.at[pl.ds(0, SGW)], sem_s0)
    d_d0 = pltpu.async_copy(dst_hbm.at[wid, pl.ds(0, SG)],
                            dstr_v.at[pl.ds(0, SG)], sem_s0)
    d_cnt = pltpu.async_copy(cnt_hbm.at[wid], cnt_v, sem_s1)
    zeros16f = jnp.zeros((16,), jnp.float32)

    def zz(i, carry):
        rows0[i // 8, pl.ds((i % 8) * 16, 16)] = zeros16f
        return carry
    lax.fori_loop(0, 128 * 8, zz, 0)

    zd = [pltpu.async_copy(rows0, agg_sh.at[pl.ds(s * (NPAD // NS) + j * 128, 128)], sem_z)
          for j in range(NPAD // NS // 128)]
    d_cnt.wait()
    cnt = cnt_v[...][0]
    nch = lax.div(cnt + 127, jnp.int32(128))
    nsg_live = lax.div(nch + (SG - 1), jnp.int32(SG))

    @pl.when(nsg_live > 1)
    def _():
        pltpu.async_copy(src_hbm.at[wid, pl.ds(SGW, SGW)],
                         srcr_v.at[pl.ds(SGW, SGW)], sem_s1)
        pltpu.async_copy(dst_hbm.at[wid, pl.ds(SG, SG)],
                         dstr_v.at[pl.ds(SG, SG)], sem_s1)
    d_s0.wait()
    d_d0.wait()
    for d in zd:
        d.wait()
    plsc.subcore_barrier()

    for b in range(NBUF):
        @pl.when(b < nch)
        def _(b=b):
            pltpu.async_copy(
                hs_hbm.at[srcr_v.at[pl.ds(b * 128, 128)]], rows[b], sems[b])

    def sgrp(sg, carry):
        par = lax.rem(sg, jnp.int32(2))
        nxt = 1 - par
        base = par * SGW
        dbase = par * SG

        @pl.when(sg > 0)
        def _():
            pltpu.make_async_copy(src_hbm.at[wid, pl.ds(0, SGW)],
                                  srcr_v.at[pl.ds(0, SGW)], sem_s1).wait()
            pltpu.make_async_copy(dst_hbm.at[wid, pl.ds(0, SG)],
                                  dstr_v.at[pl.ds(0, SG)], sem_s1).wait()
            for b in range(NBUF):
                @pl.when(sg * SG + b < nch)
                def _(b=b):
                    pltpu.async_copy(
                        hs_hbm.at[srcr_v.at[pl.ds(base + b * 128, 128)]],
                        rows[b], sems[b])

        @pl.when(jnp.logical_and(sg > 0, sg + 1 < nsg_live))
        def _():
            pltpu.async_copy(src_hbm.at[wid, pl.ds((sg + 1) * SGW, SGW)],
                             srcr_v.at[pl.ds(nxt * SGW, SGW)], sem_s1)
            pltpu.async_copy(dst_hbm.at[wid, pl.ds((sg + 1) * SG, SG)],
                             dstr_v.at[pl.ds(nxt * SG, SG)], sem_s1)

        for b8 in range(SG):
            bb = b8 % NBUF

            @pl.when(sg * SG + b8 < nch)
            def _(b8=b8, bb=bb):
                pltpu.make_async_copy(
                    hs_hbm.at[srcr_v.at[pl.ds(0, 128)]], rows[bb], sems[bb]).wait()
                pltpu.sync_copy(rows[bb], agg_sh.at[dstr_v.at[dbase + b8]], add=True)
                if b8 + NBUF < SG:
                    @pl.when(sg * SG + b8 + NBUF < nch)
                    def _():
                        pltpu.async_copy(
                            hs_hbm.at[srcr_v.at[pl.ds(base + (b8 + NBUF) * 128, 128)]],
                            rows[bb], sems[bb])
        return carry
    lax.fori_loop(0, nsg_live, sgrp, 0)
    plsc.subcore_barrier()

    pltpu.sync_copy(agg_sh.at[pl.ds(s * (NPAD // NS), NPAD // NS)],
                    agg_hbm.at[c, pl.ds(s * (NPAD // NS), NPAD // NS)])


_agg = pl.kernel(
    _agg_body,
    out_type=[jax.ShapeDtypeStruct((NC, NPAD, 128), jnp.float32)],
    mesh=plsc.VectorSubcoreMesh(core_axis_name="c", subcore_axis_name="s"),
    scratch_types=[
        pltpu.VMEM((2 * SG * 128,), jnp.int32),
        pltpu.VMEM((2 * SG, 128), jnp.int32),
        pltpu.VMEM((16,), jnp.int32),
        pltpu.VMEM((128, 128), jnp.float32),
        pltpu.VMEM((128, 128), jnp.float32),
        pltpu.VMEM_SHARED((NPAD, 128), jnp.float32),
        pltpu.SemaphoreType.DMA,
        pltpu.SemaphoreType.DMA,
        pltpu.SemaphoreType.DMA,
        pltpu.SemaphoreType.DMA,
        pltpu.SemaphoreType.DMA,
    ],
    compiler_params=pltpu.CompilerParams(needs_layout_passes=False),
)


def kernel(x, inp_c, edge_index, batch, W1, b1, W2, b2, W3, b3, p1, p2, p3, We, L1, bL1, L2, bL2, L3):
    src0 = edge_index[0].astype(jnp.int32).reshape(NW, E // NW)
    dst0 = edge_index[1].astype(jnp.int32).reshape(NW, E // NW)
    xpad = jnp.zeros((NPAD, H), x.dtype).at[:N].set(x)
    batch_pad = jnp.concatenate([batch.astype(jnp.int32), jnp.full((NPAD - N,), NG - 1, jnp.int32)])
    batcht = batch_pad.reshape(1, NPAD)
    gids = jnp.arange(NG, dtype=batch_pad.dtype)
    starts = jnp.searchsorted(batch_pad, gids, side="left").astype(jnp.int32)
    ends = jnp.searchsorted(batch_pad, gids, side="right").astype(jnp.int32)
    sel0_2d = (lax.iota(jnp.int32, NPAD) < N).astype(jnp.int32).reshape(NROW, 128)

    def layer(g, sel2d, selrow, src_c, dst_c, W, b, p, selfn, first):
        cd = _compact_deg_l1 if first else _compact_deg_l2
        srco, dsto, dego, cnts = cd(src_c, dst_c, sel2d)
        hs, dinv = _mm(g, W, dego.reshape(2, NPAD, 1))
        agg2 = _agg(srco, dsto.reshape(NW, ECT // 128, 128), cnts, hs)[0]
        gnext, key = _epi(agg2, hs, dinv, b.reshape(1, H), p.reshape(H, 1), selrow)
        sel2d_n = selfn(key.reshape(NROW, 128))
        selrow_n = sel2d_n.reshape(NPAD, 1)
        xl = _readout(gnext, selrow_n, batcht, starts, ends)
        return gnext, sel2d_n, selrow_n, srco, dsto, xl

    sel0row = sel0_2d.reshape(NPAD, 1)
    g1, s1, s1r, src1, dst1, x1 = layer(xpad, sel0_2d, sel0row, src0, dst0, W1, b1, p1, _select_1, True)
    g2, s2, s2r, src2, dst2, x2 = layer(g1, s1, s1r, src1, dst1, W2, b2, p2, _select_2, False)
    g3, s3, s3r, src3, dst3, x3 = layer(g2, s2, s2r, src2, dst2, W3, b3, p3, _select_3, False)
    return _head(x1, x2, x3, inp_c, We, L1, bL1, L2, bL2, L3)


# speedup vs baseline: 1.0250x; 1.0250x over previous
import functools

import jax
import jax.numpy as jnp
from jax import lax
from jax.experimental import pallas as pl
from jax.experimental.pallas import tpu as pltpu
from jax.experimental.pallas import tpu_sc as plsc

N = 10000
E = 320000
NPAD = 10240
SENT = 10000
NG = 64
H = 128
NC = 2
NS = 16
NW = NC * NS
ECT = 10240
NCH_OUT = ECT // 16
NROW = NPAD // 128


BN = 1024


def _mm_body(g_ref, w_ref, deg_ref, hs_ref, dinv_ref):
    degv = deg_ref[...]
    deg = degv[0] + degv[1] + 1.0
    dinv = lax.rsqrt(deg)
    gs = g_ref[...] * dinv
    hs_ref[...] = jnp.dot(gs, w_ref[...], preferred_element_type=jnp.float32)
    dinv_ref[...] = dinv


def _mm(g, w, deg2):
    return pl.pallas_call(
        _mm_body,
        grid=(NPAD // BN,),
        in_specs=[
            pl.BlockSpec((BN, H), lambda i: (i, 0)),
            pl.BlockSpec((H, H), lambda i: (0, 0)),
            pl.BlockSpec((2, BN, 1), lambda i: (0, i, 0)),
        ],
        out_specs=[
            pl.BlockSpec((BN, H), lambda i: (i, 0)),
            pl.BlockSpec((BN, 1), lambda i: (i, 0)),
        ],
        out_shape=[
            jax.ShapeDtypeStruct((NPAD, H), jnp.float32),
            jax.ShapeDtypeStruct((NPAD, 1), jnp.float32),
        ],
    )(g, w, deg2)


def _epi_body(agg_ref, hs_ref, dinv_ref, b_ref, p_ref, sel_ref, gn_ref, key_ref):
    aggv = agg_ref[...]
    hs = hs_ref[...]
    dinv = dinv_ref[...]
    h = jax.nn.relu(dinv * (aggv[0] + aggv[1] + hs) + b_ref[...])
    pc = p_ref[...]
    norm = jnp.sqrt(jnp.sum(pc * pc))
    sc = jnp.tanh(jnp.dot(h, pc, preferred_element_type=jnp.float32) / (norm + 1e-12))
    gn_ref[...] = h * sc
    key_ref[...] = jnp.where(sel_ref[...] > 0, sc, -2.0)


def _epi(agg2, hs, dinv, b2d, pcol, selrow):
    return pl.pallas_call(
        _epi_body,
        grid=(NPAD // BN,),
        in_specs=[
            pl.BlockSpec((2, BN, H), lambda i: (0, i, 0)),
            pl.BlockSpec((BN, H), lambda i: (i, 0)),
            pl.BlockSpec((BN, 1), lambda i: (i, 0)),
            pl.BlockSpec((1, H), lambda i: (0, 0)),
            pl.BlockSpec((H, 1), lambda i: (0, 0)),
            pl.BlockSpec((BN, 1), lambda i: (i, 0)),
        ],
        out_specs=[
            pl.BlockSpec((BN, H), lambda i: (i, 0)),
            pl.BlockSpec((BN, 1), lambda i: (i, 0)),
        ],
        out_shape=[
            jax.ShapeDtypeStruct((NPAD, H), jnp.float32),
            jax.ShapeDtypeStruct((NPAD, 1), jnp.float32),
        ],
    )(agg2, hs, dinv, b2d, pcol, selrow)


def _make_select(k):
    def body(key_ref, sel_ref):
        keyv = key_ref[...]
        bu = lax.bitcast_convert_type(keyv, jnp.uint32)
        top = jnp.uint32(0x80000000)
        mu = jnp.where(bu >= top, jnp.bitwise_not(bu), bu | top)

        def sb(i, t):
            sh = (31 - i).astype(jnp.uint32)
            cand = t | lax.shift_left(jnp.uint32(1), sh)
            cnt = jnp.sum((mu >= cand).astype(jnp.int32))
            return jnp.where(cnt >= k, cand, t)

        t = lax.fori_loop(0, 32, sb, jnp.uint32(0))
        gt = mu > t
        tie = (mu == t).astype(jnp.float32)
        need = (k - jnp.sum(gt.astype(jnp.int32))).astype(jnp.float32)
        m128 = (lax.broadcasted_iota(jnp.int32, (128, 128), 0)
                <= lax.broadcasted_iota(jnp.int32, (128, 128), 1)).astype(jnp.float32)
        rank_in_row = jnp.dot(tie, m128, preferred_element_type=jnp.float32)
        rowsum = jnp.sum(tie, axis=1, keepdims=True)
        s80 = (lax.broadcasted_iota(jnp.int32, (NROW, NROW), 1)
               < lax.broadcasted_iota(jnp.int32, (NROW, NROW), 0)).astype(jnp.float32)
        rowoff = jnp.dot(s80, rowsum, preferred_element_type=jnp.float32)
        rank = rank_in_row + rowoff
        selv = gt | ((tie > 0.0) & (rank <= need))
        sel_ref[...] = selv.astype(jnp.int32)

    return pl.pallas_call(
        body,
        out_shape=jax.ShapeDtypeStruct((NROW, 128), jnp.int32),
    )


_select_1 = _make_select(5000)
_select_2 = _make_select(2500)
_select_3 = _make_select(1250)

NBLK = NPAD // 8


def _readout_body(g_ref, sel_ref, btt_ref, st_ref, en_ref, xl_ref, gm_s):
    neg = jnp.float32(-jnp.inf)
    sel = sel_ref[...] > 0
    g = g_ref[...]
    gm = jnp.where(sel, g, neg)
    gm_s[...] = gm
    bm = jnp.max(gm.reshape(NBLK, 8, 128), axis=1)
    stt0 = (btt_ref[...] == lax.broadcasted_iota(jnp.int32, (NG, 1), 0)).astype(jnp.float32)
    gs = jnp.where(sel, g, 0.0)
    sums = jnp.dot(stt0, gs, preferred_element_type=jnp.float32)
    counts = jnp.dot(stt0, sel.astype(jnp.float32), preferred_element_type=jnp.float32)
    ga = sums / jnp.maximum(counts, 1.0)
    xl_ref[pl.ds(0, NG), pl.ds(128, 128)] = ga
    biota = lax.broadcasted_iota(jnp.int32, (NBLK, 1), 0)
    riota8 = lax.broadcasted_iota(jnp.int32, (8, 1), 0)
    giota = lax.broadcasted_iota(jnp.int32, (NG, 1), 0)

    def gloop(gi, acc):
        s_g = st_ref[gi]
        e_g = en_ref[gi]
        bs = (s_g + 7) // 8
        be = e_g // 8
        m = (biota >= bs) & (biota < be)
        mfull = jnp.max(jnp.where(m, bm, neg), axis=0, keepdims=True)
        hb = s_g // 8
        tb = jnp.maximum(e_g - 1, 0) // 8
        hrows = gm_s[pl.ds(hb * 8, 8), :]
        hm = (riota8 + hb * 8 >= s_g) & (riota8 + hb * 8 < e_g)
        hmax = jnp.max(jnp.where(hm, hrows, neg), axis=0, keepdims=True)
        trows = gm_s[pl.ds(tb * 8, 8), :]
        tm = (riota8 + tb * 8 >= s_g) & (riota8 + tb * 8 < e_g)
        tmax = jnp.max(jnp.where(tm, trows, neg), axis=0, keepdims=True)
        mg = jnp.maximum(jnp.maximum(mfull, hmax), tmax)
        mg = jnp.where(jnp.isfinite(mg), mg, 0.0)
        return jnp.where(giota == gi, mg, acc)

    gmax = lax.fori_loop(0, NG, gloop, jnp.zeros((NG, 128), jnp.float32))
    xl_ref[pl.ds(0, NG), pl.ds(0, 128)] = gmax


def _readout(gnext, selrow, batcht, starts, ends):
    return pl.pallas_call(
        _readout_body,
        in_specs=[
            pl.BlockSpec((NPAD, H), lambda: (0, 0)),
            pl.BlockSpec((NPAD, 1), lambda: (0, 0)),
            pl.BlockSpec((1, NPAD), lambda: (0, 0)),
            pl.BlockSpec(memory_space=pltpu.SMEM),
            pl.BlockSpec(memory_space=pltpu.SMEM),
        ],
        out_shape=jax.ShapeDtypeStruct((NG, 256), jnp.float32),
        scratch_shapes=[
            pltpu.VMEM((NPAD, 128), jnp.float32),
        ],
    )(gnext, selrow, batcht, starts, ends)


def _head_body(x1_ref, x2_ref, x3_ref, ic_ref, we_ref, l1_ref, bl1_ref,
               l2_ref, bl2_ref, l3_ref, o_ref):
    hg = (jax.nn.relu(x1_ref[...]) + jax.nn.relu(x2_ref[...])
          + jax.nn.relu(x3_ref[...]))
    emb = jax.nn.relu(jnp.dot(ic_ref[...], we_ref[...], preferred_element_type=jnp.float32))
    fus = jnp.concatenate([emb, hg], axis=1)
    o = jax.nn.relu(jnp.dot(fus, l1_ref[...], preferred_element_type=jnp.float32) + bl1_ref[...])
    o = jax.nn.relu(jnp.dot(o, l2_ref[...], preferred_element_type=jnp.float32) + bl2_ref[...])
    o_ref[...] = jnp.dot(o, l3_ref[...], preferred_element_type=jnp.float32)


def _head(x1, x2, x3, inp_c, We, L1, bL1, L2, bL2, L3):
    return pl.pallas_call(
        _head_body,
        out_shape=jax.ShapeDtypeStruct((NG, 1), jnp.float32),
    )(x1, x2, x3, inp_c, We, L1, bL1.reshape(1, -1), L2, bL2.reshape(1, -1), L3)


def _make_compact_deg(ein_t):
    nch_in = ein_t // 16
    mesh = plsc.VectorSubcoreMesh(core_axis_name="c", subcore_axis_name="s")

    def body(src_hbm, dst_hbm, sel_hbm, srco_hbm, dsto_hbm, deg_hbm, cnt_hbm,
             sel_v, src_v, dst_v, srco_v, dsto_v, deg_v, idx80_v, cnt_v, deg_sh):
        c = lax.axis_index("c")
        s = lax.axis_index("s")
        wid = c * NS + s
        pltpu.sync_copy(sel_hbm, sel_v)
        pltpu.sync_copy(src_hbm.at[wid], src_v)
        pltpu.sync_copy(dst_hbm.at[wid], dst_v)
        zeros16f = jnp.zeros((16,), jnp.float32)
        sent16 = jnp.full((16,), SENT, jnp.int32)
        ones16f = jnp.ones((16,), jnp.float32)
        c127 = jnp.full((16,), 127, jnp.int32)

        def z_body(i, carry):
            deg_v[i // 8, pl.ds((i % 8) * 16, 16)] = zeros16f
            return carry
        lax.fori_loop(0, NROW * 8, z_body, 0)

        def f_body(i, carry):
            srco_v[pl.ds(i * 16, 16)] = sent16
            dsto_v[pl.ds(i * 16, 16)] = sent16
            return carry
        lax.fori_loop(0, NCH_OUT, f_body, 0)

        def i_body(j, carry):
            idx80_v[pl.ds(j * 16, 16)] = lax.iota(jnp.int32, 16) + j * 16
            return carry
        lax.fori_loop(0, NROW // 16, i_body, 0)

        def e_body(i, off):
            s16 = src_v[pl.ds(i * 16, 16)]
            d16 = dst_v[pl.ds(i * 16, 16)]
            ss = plsc.load_gather(sel_v, [lax.shift_right_logical(s16, 7), lax.bitwise_and(s16, c127)])
            dd = plsc.load_gather(sel_v, [lax.shift_right_logical(d16, 7), lax.bitwise_and(d16, c127)])
            m = (ss > 0) & (dd > 0)
            plsc.store_compressed(srco_v.at[pl.ds(off, 16)], s16, mask=m)
            plsc.store_compressed(dsto_v.at[pl.ds(off, 16)], d16, mask=m)
            r16 = lax.shift_right_logical(d16, 7)
            c16 = lax.bitwise_and(d16, c127)
            plsc.addupdate_scatter(deg_v, [r16, c16], ones16f, mask=m)
            return off + jnp.sum(m.astype(jnp.int32))
        off = lax.fori_loop(0, nch_in, e_body, jnp.int32(0))

        cnt_v[...] = jnp.full((16,), 1, jnp.int32) * off
        pltpu.sync_copy(cnt_v, cnt_hbm.at[wid])
        pltpu.sync_copy(srco_v, srco_hbm.at[wid])
        pltpu.sync_copy(dsto_v, dsto_hbm.at[wid])

        @pl.when(s == 0)
        def _():
            pltpu.sync_copy(deg_v, deg_sh)
        plsc.subcore_barrier()

        @pl.when(s != 0)
        def _():
            pltpu.sync_copy(deg_v, deg_sh.at[idx80_v], add=True)
        plsc.subcore_barrier()

        @pl.when(s == 0)
        def _():
            pltpu.sync_copy(deg_sh, deg_hbm.at[c])

    return pl.kernel(
        body,
        out_type=[
            jax.ShapeDtypeStruct((NW, ECT), jnp.int32),
            jax.ShapeDtypeStruct((NW, ECT), jnp.int32),
            jax.ShapeDtypeStruct((NC, NROW, 128), jnp.float32),
            jax.ShapeDtypeStruct((NW, 16), jnp.int32),
        ],
        mesh=mesh,
        scratch_types=[
            pltpu.VMEM((NROW, 128), jnp.int32),
            pltpu.VMEM((ein_t,), jnp.int32),
            pltpu.VMEM((ein_t,), jnp.int32),
            pltpu.VMEM((ECT,), jnp.int32),
            pltpu.VMEM((ECT,), jnp.int32),
            pltpu.VMEM((NROW, 128), jnp.float32),
            pltpu.VMEM((NROW,), jnp.int32),
            pltpu.VMEM((16,), jnp.int32),
            pltpu.VMEM_SHARED((NROW, 128), jnp.float32),
        ],
        compiler_params=pltpu.CompilerParams(needs_layout_passes=False),
    )


_compact_deg_l1 = _make_compact_deg(E // NW)
_compact_deg_l2 = _make_compact_deg(ECT)


NBUF = 2
SG = 16
NSG = ECT // (SG * 128)


def _agg_body(src_hbm, dst_hbm, cnt_hbm, hs_hbm, agg_hbm,
              srcr_v, dstr_v, cnt_v, rows0, rows1, agg_sh,
              sem_s0, sem_s1, sem0, sem1, sem_z):
    c = lax.axis_index("c")
    s = lax.axis_index("s")
    wid = c * NS + s
    rows = (rows0, rows1)
    sems = (sem0, sem1)
    ssems = (sem_s0, sem_s1)
    SGW = SG * 128
    d_s0 = pltpu.async_copy(src_hbm.at[wid, pl.ds(0, SGW)],
                            srcr_v.at[pl.ds(0, SGW)], sem_s0)
    d_d0 = pltpu.async_copy(dst_hbm.at[wid, pl.ds(0, SG)],
                            dstr_v.at[pl.ds(0, SG)], sem_s0)
    d_cnt = pltpu.async_copy(cnt_hbm.at[wid], cnt_v, sem_s1)
    zeros16f = jnp.zeros((16,), jnp.float32)

    def zz(i, carry):
        rows0[i // 8, pl.ds((i % 8) * 16, 16)] = zeros16f
        return carry
    lax.fori_loop(0, 128 * 8, zz, 0)

    zd = [pltpu.async_copy(rows0, agg_sh.at[pl.ds(s * (NPAD // NS) + j * 128, 128)], sem_z)
          for j in range(NPAD // NS // 128)]
    d_cnt.wait()
    cnt = cnt_v[...][0]
    nch = lax.div(cnt + 127, jnp.int32(128))
    nsg_live = lax.div(nch + (SG - 1), jnp.int32(SG))

    @pl.when(nsg_live > 1)
    def _():
        pltpu.async_copy(src_hbm.at[wid, pl.ds(SGW, SGW)],
                         srcr_v.at[pl.ds(SGW, SGW)], sem_s1)
        pltpu.async_copy(dst_hbm.at[wid, pl.ds(SG, SG)],
                         dstr_v.at[pl.ds(SG, SG)], sem_s1)
    d_s0.wait()
    d_d0.wait()
    for d in zd:
        d.wait()
    plsc.subcore_barrier()

    for b in range(NBUF):
        @pl.when(b < nch)
        def _(b=b):
            pltpu.async_copy(
                hs_hbm.at[srcr_v.at[pl.ds(b * 128, 128)]], rows[b], sems[b])

    def sgrp(sg, carry):
        par = lax.rem(sg, jnp.int32(2))
        nxt = 1 - par
        base = par * SGW
        dbase = par * SG

        @pl.when(sg > 0)
        def _():
            pltpu.make_async_copy(src_hbm.at[wid, pl.ds(0, SGW)],
                                  srcr_v.at[pl.ds(0, SGW)], sem_s1).wait()
            pltpu.make_async_copy(dst_hbm.at[wid, pl.ds(0, SG)],
                                  dstr_v.at[pl.ds(0, SG)], sem_s1).wait()
            for b in range(NBUF):
                @pl.when(sg * SG + b < nch)
                def _(b=b):
                    pltpu.async_copy(
                        hs_hbm.at[srcr_v.at[pl.ds(base + b * 128, 128)]],
                        rows[b], sems[b])

        @pl.when(jnp.logical_and(sg > 0, sg + 1 < nsg_live))
        def _():
            pltpu.async_copy(src_hbm.at[wid, pl.ds((sg + 1) * SGW, SGW)],
                             srcr_v.at[pl.ds(nxt * SGW, SGW)], sem_s1)
            pltpu.async_copy(dst_hbm.at[wid, pl.ds((sg + 1) * SG, SG)],
                             dstr_v.at[pl.ds(nxt * SG, SG)], sem_s1)

        for b8 in range(SG):
            bb = b8 % NBUF

            @pl.when(sg * SG + b8 < nch)
            def _(b8=b8, bb=bb):
                pltpu.make_async_copy(
                    hs_hbm.at[srcr_v.at[pl.ds(0, 128)]], rows[bb], sems[bb]).wait()
                pltpu.sync_copy(rows[bb], agg_sh.at[dstr_v.at[dbase + b8]], add=True)
                if b8 + NBUF < SG:
                    @pl.when(sg * SG + b8 + NBUF < nch)
                    def _():
                        pltpu.async_copy(
                            hs_hbm.at[srcr_v.at[pl.ds(base + (b8 + NBUF) * 128, 128)]],
                            rows[bb], sems[bb])
        return carry
    lax.fori_loop(0, nsg_live, sgrp, 0)
    plsc.subcore_barrier()

    pltpu.sync_copy(agg_sh.at[pl.ds(s * (NPAD // NS), NPAD // NS)],
                    agg_hbm.at[c, pl.ds(s * (NPAD // NS), NPAD // NS)])


_agg = pl.kernel(
    _agg_body,
    out_type=[jax.ShapeDtypeStruct((NC, NPAD, 128), jnp.float32)],
    mesh=plsc.VectorSubcoreMesh(core_axis_name="c", subcore_axis_name="s"),
    scratch_types=[
        pltpu.VMEM((2 * SG * 128,), jnp.int32),
        pltpu.VMEM((2 * SG, 128), jnp.int32),
        pltpu.VMEM((16,), jnp.int32),
        pltpu.VMEM((128, 128), jnp.float32),
        pltpu.VMEM((128, 128), jnp.float32),
        pltpu.VMEM_SHARED((NPAD, 128), jnp.float32),
        pltpu.SemaphoreType.DMA,
        pltpu.SemaphoreType.DMA,
        pltpu.SemaphoreType.DMA,
        pltpu.SemaphoreType.DMA,
        pltpu.SemaphoreType.DMA,
    ],
    compiler_params=pltpu.CompilerParams(needs_layout_passes=False),
)


def kernel(x, inp_c, edge_index, batch, W1, b1, W2, b2, W3, b3, p1, p2, p3, We, L1, bL1, L2, bL2, L3):
    src0 = edge_index[0].astype(jnp.int32).reshape(NW, E // NW)
    dst0 = edge_index[1].astype(jnp.int32).reshape(NW, E // NW)
    xpad = jnp.zeros((NPAD, H), x.dtype).at[:N].set(x)
    batch_pad = jnp.concatenate([batch.astype(jnp.int32), jnp.full((NPAD - N,), NG - 1, jnp.int32)])
    batcht = batch_pad.reshape(1, NPAD)
    gids = jnp.arange(NG, dtype=batch_pad.dtype)
    starts = jnp.searchsorted(batch_pad, gids, side="left").astype(jnp.int32)
    ends = jnp.searchsorted(batch_pad, gids, side="right").astype(jnp.int32)
    sel0_2d = (lax.iota(jnp.int32, NPAD) < N).astype(jnp.int32).reshape(NROW, 128)

    def layer(g, sel2d, selrow, src_c, dst_c, W, b, p, selfn, first):
        cd = _compact_deg_l1 if first else _compact_deg_l2
        srco, dsto, dego, cnts = cd(src_c, dst_c, sel2d)
        hs, dinv = _mm(g, W, dego.reshape(2, NPAD, 1))
        agg2 = _agg(srco, dsto.reshape(NW, ECT // 128, 128), cnts, hs)[0]
        gnext, key = _epi(agg2, hs, dinv, b.reshape(1, H), p.reshape(H, 1), selrow)
        sel2d_n = selfn(key.reshape(NROW, 128))
        selrow_n = sel2d_n.reshape(NPAD, 1)
        xl = _readout(gnext, selrow_n, batcht, starts, ends)
        return gnext, sel2d_n, selrow_n, srco, dsto, xl

    sel0row = sel0_2d.reshape(NPAD, 1)
    g1, s1, s1r, src1, dst1, x1 = layer(xpad, sel0_2d, sel0row, src0, dst0, W1, b1, p1, _select_1, True)
    g2, s2, s2r, src2, dst2, x2 = layer(g1, s1, s1r, src1, dst1, W2, b2, p2, _select_2, False)
    g3, s3, s3r, src3, dst3, x3 = layer(g2, s2, s2r, src2, dst2, W3, b3, p3, _select_3, False)
    return _head(x1, x2, x3, inp_c, We, L1, bL1, L2, bL2, L3)
